# Initial kernel scaffold; baseline (speedup 1.0000x reference)
import jax
import jax.numpy as jnp
from jax.experimental import pallas as pl


def kernel(node_attr, edge_index, edge_attr, edge_world_index, edge_world_attr, W, b):
    raise NotImplementedError("write your pallas kernel here")


# speedup vs baseline: 3.1164x; 3.1164x over previous
import functools

import jax
import jax.numpy as jnp
from jax import lax
from jax.experimental import pallas as pl
from jax.experimental.pallas import tpu as pltpu
from jax.experimental.pallas import tpu_sc as plsc

_N = 10000
_E = 320000
_D = 128
_DE = 16

_NC = 2
_NS = 16
_NW = _NC * _NS
_EPW = _E // _NW
_C = 80
_NCH = _EPW // _C


def _proj_body(node_ref, ws_ref, wr_ref, b_ref, ps_ref, pr_ref):
    n = node_ref[...]
    ps_ref[...] = (
        jnp.dot(n, ws_ref[...], preferred_element_type=jnp.float32) + b_ref[...]
    )
    pr_ref[...] = jnp.dot(n, wr_ref[...], preferred_element_type=jnp.float32)


def _node_projections(node_attr, w_s, w_r, b2):
    return pl.pallas_call(
        _proj_body,
        out_shape=[
            jax.ShapeDtypeStruct((_N, _D), jnp.float32),
            jax.ShapeDtypeStruct((_N, _D), jnp.float32),
        ],
    )(node_attr, w_s, w_r, b2)


def _gather_sum_body(ps_hbm, pr_hbm, s_hbm, r_hbm, out_hbm,
                     sidx_v, ridx_v, rows_s, rows_r, sem_s, sem_r):
    widx = lax.axis_index("s") * _NC + lax.axis_index("c")
    ebase = widx * _EPW
    pltpu.sync_copy(s_hbm.at[pl.ds(ebase, _EPW)], sidx_v)
    pltpu.sync_copy(r_hbm.at[pl.ds(ebase, _EPW)], ridx_v)

    def chunk_body(i, _):
        cbase = i * _C
        cp_s = pltpu.async_copy(
            ps_hbm.at[sidx_v.at[pl.ds(cbase, _C)]], rows_s, sem_s)
        cp_r = pltpu.async_copy(
            pr_hbm.at[ridx_v.at[pl.ds(cbase, _C)]], rows_r, sem_r)
        cp_s.wait()
        cp_r.wait()

        def edge_body(e, carry):
            for j in range(_D // 16):
                sl = pl.ds(j * 16, 16)
                rows_s[e, sl] = rows_s[e, sl] + rows_r[e, sl]
            return carry

        lax.fori_loop(0, _C, edge_body, 0)
        pltpu.sync_copy(rows_s, out_hbm.at[pl.ds(ebase + cbase, _C)])
        return 0

    lax.fori_loop(0, _NCH, chunk_body, 0)


def _gather_sum(p_s, p_r, s_idx, r_idx):
    mesh = plsc.VectorSubcoreMesh(core_axis_name="c", subcore_axis_name="s")
    k = functools.partial(
        pl.kernel,
        mesh=mesh,
        out_type=jax.ShapeDtypeStruct((_E, _D), jnp.float32),
        scratch_types=[
            pltpu.VMEM((_EPW,), jnp.int32),
            pltpu.VMEM((_EPW,), jnp.int32),
            pltpu.VMEM((_C, _D), jnp.float32),
            pltpu.VMEM((_C, _D), jnp.float32),
            pltpu.SemaphoreType.DMA,
            pltpu.SemaphoreType.DMA,
        ],
    )(_gather_sum_body)
    return k(p_s, p_r, s_idx, r_idx)


_BLK = 2000


def _edge_out_body(g_ref, ewa_ref, we_ref, out_ref):
    out_ref[...] = g_ref[...] + jnp.dot(
        ewa_ref[...], we_ref[...], preferred_element_type=jnp.float32)


def _edge_out(g, ewa, w_e):
    return pl.pallas_call(
        _edge_out_body,
        grid=(_E // _BLK,),
        in_specs=[
            pl.BlockSpec((_BLK, _D), lambda i: (i, 0)),
            pl.BlockSpec((_BLK, _DE), lambda i: (i, 0)),
            pl.BlockSpec((_DE, _D), lambda i: (0, 0)),
        ],
        out_specs=pl.BlockSpec((_BLK, _D), lambda i: (i, 0)),
        out_shape=jax.ShapeDtypeStruct((_E, _D), jnp.float32),
    )(g, ewa, w_e)


def kernel(node_attr, edge_index, edge_attr, edge_world_index, edge_world_attr, W, b):
    w_s = W[:_D]
    w_r = W[_D:2 * _D]
    w_e = W[2 * _D:]
    b2 = b.reshape(1, _D)
    s_idx = edge_world_index[0]
    r_idx = edge_world_index[1]

    p_s, p_r = _node_projections(node_attr, w_s, w_r, b2)
    g = _gather_sum(p_s, p_r, s_idx, r_idx)
    new_edge_world_attr = _edge_out(g, edge_world_attr, w_e)
    return (node_attr, edge_attr, edge_index, edge_world_index, new_edge_world_attr)


# speedup vs baseline: 3.6275x; 1.1640x over previous
import functools

import jax
import jax.numpy as jnp
from jax import lax
from jax.experimental import pallas as pl
from jax.experimental.pallas import tpu as pltpu
from jax.experimental.pallas import tpu_sc as plsc

_N = 10000
_E = 320000
_D = 128
_DE = 16

_NC = 2
_NS = 16
_NW = _NC * _NS
_EPW = _E // _NW
_C = 40
_NCH = _EPW // _C


def _proj_body(node_ref, ws_ref, wr_ref, b_ref, ps_ref, pr_ref):
    n = node_ref[...]
    ps_ref[...] = (
        jnp.dot(n, ws_ref[...], preferred_element_type=jnp.float32) + b_ref[...]
    )
    pr_ref[...] = jnp.dot(n, wr_ref[...], preferred_element_type=jnp.float32)


def _node_projections(node_attr, w_s, w_r, b2):
    return pl.pallas_call(
        _proj_body,
        out_shape=[
            jax.ShapeDtypeStruct((_N, _D), jnp.float32),
            jax.ShapeDtypeStruct((_N, _D), jnp.float32),
        ],
    )(node_attr, w_s, w_r, b2)


def _gather_sum_body(ps_hbm, pr_hbm, s_hbm, r_hbm, out_hbm,
                     sidx, ridx,
                     rs0, rr0, ro0, rs1, rr1, ro1,
                     sem_s0, sem_r0, sem_w0, sem_s1, sem_r1, sem_w1):
    rs = (rs0, rs1)
    rr = (rr0, rr1)
    ro = (ro0, ro1)
    sem_s = (sem_s0, sem_s1)
    sem_r = (sem_r0, sem_r1)
    sem_w = (sem_w0, sem_w1)

    widx = lax.axis_index("s") * _NC + lax.axis_index("c")
    ebase = widx * _EPW
    pltpu.sync_copy(s_hbm.at[pl.ds(ebase, _EPW)], sidx)
    pltpu.sync_copy(r_hbm.at[pl.ds(ebase, _EPW)], ridx)

    def issue_gathers(ci, b):
        cb = ci * _C
        pltpu.async_copy(ps_hbm.at[sidx.at[pl.ds(cb, _C)]], rs[b], sem_s[b])
        pltpu.async_copy(pr_hbm.at[ridx.at[pl.ds(cb, _C)]], rr[b], sem_r[b])

    def wait_gathers(ci, b):
        cb = ci * _C
        pltpu.make_async_copy(
            ps_hbm.at[sidx.at[pl.ds(cb, _C)]], rs[b], sem_s[b]).wait()
        pltpu.make_async_copy(
            pr_hbm.at[ridx.at[pl.ds(cb, _C)]], rr[b], sem_r[b]).wait()

    def out_slice(ci):
        return out_hbm.at[pl.ds(ebase + ci * _C, _C)]

    issue_gathers(0, 0)
    issue_gathers(1, 1)

    def round_body(g, carry):
        for b in range(2):
            ci = 2 * g + b
            wait_gathers(ci, b)

            @pl.when(g > 0)
            def _():
                pltpu.make_async_copy(ro[b], out_slice(ci - 2), sem_w[b]).wait()

            def edge_body(e, acc):
                for j in range(_D // 16):
                    sl = pl.ds(j * 16, 16)
                    ro[b][e, sl] = rs[b][e, sl] + rr[b][e, sl]
                return acc

            lax.fori_loop(0, _C, edge_body, 0)
            pltpu.async_copy(ro[b], out_slice(ci), sem_w[b])

            @pl.when(ci + 2 < _NCH)
            def _():
                issue_gathers(ci + 2, b)
        return carry

    lax.fori_loop(0, _NCH // 2, round_body, 0)
    pltpu.make_async_copy(ro[0], out_slice(_NCH - 2), sem_w[0]).wait()
    pltpu.make_async_copy(ro[1], out_slice(_NCH - 1), sem_w[1]).wait()


def _gather_sum(p_s, p_r, s_idx, r_idx):
    mesh = plsc.VectorSubcoreMesh(core_axis_name="c", subcore_axis_name="s")
    k = functools.partial(
        pl.kernel,
        mesh=mesh,
        out_type=jax.ShapeDtypeStruct((_E, _D), jnp.float32),
        scratch_types=[
            pltpu.VMEM((_EPW,), jnp.int32),
            pltpu.VMEM((_EPW,), jnp.int32),
        ] + [pltpu.VMEM((_C, _D), jnp.float32)] * 6
          + [pltpu.SemaphoreType.DMA] * 6,
    )(_gather_sum_body)
    return k(p_s, p_r, s_idx, r_idx)


_BLK = 2000


def _edge_out_body(g_ref, ewa_ref, we_ref, out_ref):
    ewa16 = ewa_ref[...].astype(jnp.bfloat16)
    we16 = we_ref[...].astype(jnp.bfloat16)
    out_ref[...] = g_ref[...] + jnp.dot(
        ewa16, we16, preferred_element_type=jnp.float32)


def _edge_out(g, ewa, w_e):
    return pl.pallas_call(
        _edge_out_body,
        grid=(_E // _BLK,),
        in_specs=[
            pl.BlockSpec((_BLK, _D), lambda i: (i, 0)),
            pl.BlockSpec((_BLK, _DE), lambda i: (i, 0)),
            pl.BlockSpec((_DE, _D), lambda i: (0, 0)),
        ],
        out_specs=pl.BlockSpec((_BLK, _D), lambda i: (i, 0)),
        out_shape=jax.ShapeDtypeStruct((_E, _D), jnp.float32),
    )(g, ewa, w_e)


def kernel(node_attr, edge_index, edge_attr, edge_world_index, edge_world_attr, W, b):
    w_s = W[:_D]
    w_r = W[_D:2 * _D]
    w_e = W[2 * _D:]
    b2 = b.reshape(1, _D)
    s_idx = edge_world_index[0]
    r_idx = edge_world_index[1]

    p_s, p_r = _node_projections(node_attr, w_s, w_r, b2)
    g = _gather_sum(p_s, p_r, s_idx, r_idx)
    new_edge_world_attr = _edge_out(g, edge_world_attr, w_e)
    return (node_attr, edge_attr, edge_index, edge_world_index, new_edge_world_attr)
